# baseline (speedup 1.0000x reference)
import functools

import jax
import jax.numpy as jnp
from jax import lax
from jax.experimental import pallas as pl
from jax.experimental.pallas import tpu as pltpu
from jax.experimental.pallas import tpu_sc as plsc

N = 10000
E = 320000
D = 128

NC = 2
NS = 16
NW = NC * NS

CHUNK = 128
NB = 4
SUB = CHUNK // NB
NCHUNKS = E // CHUNK
CHPW = 80
HALF = CHPW // 2
TAIL = NCHUNKS - (NW - 1) * CHPW
ACC_ROWS = N
WB_ROWS = 632
WB_LAST = N - (NS - 1) * WB_ROWS

DEG_PAD = 10240
DEG_SLICE = DEG_PAD // NS

_MESH = plsc.VectorSubcoreMesh(core_axis_name="c", subcore_axis_name="s")


@functools.partial(
    pl.kernel,
    out_type=jax.ShapeDtypeStruct((NC, DEG_PAD), jnp.float32),
    mesh=_MESH,
    scratch_types=[
        pltpu.VMEM_SHARED((DEG_PAD,), jnp.float32),
        pltpu.VMEM((DEG_SLICE,), jnp.float32),
        pltpu.VMEM((CHUNK,), jnp.float32),
        pltpu.VMEM((CHPW, 2, CHUNK), jnp.int32),
        pltpu.SemaphoreType.DMA,
    ],
)
def _deg_sc(e3_hbm, out_hbm, deg_sh, zbuf, ones_v, eidx, sem):
    c = lax.axis_index("c")
    s = lax.axis_index("s")
    w = c * NS + s

    @pl.when(w < NW - 1)
    def _():
        pltpu.sync_copy(e3_hbm.at[pl.ds(w * CHPW, CHPW)], eidx)

    @pl.when(w == NW - 1)
    def _():
        pltpu.sync_copy(e3_hbm.at[pl.ds((NW - 1) * CHPW, TAIL)],
                        eidx.at[pl.ds(0, TAIL)])

    def fill_z(i, _):
        zbuf[pl.ds(i * 16, 16)] = jnp.zeros((16,), jnp.float32)
        return 0

    lax.fori_loop(0, DEG_SLICE // 16, fill_z, 0)

    def fill_o(i, _):
        ones_v[pl.ds(i * 16, 16)] = jnp.ones((16,), jnp.float32)
        return 0

    lax.fori_loop(0, CHUNK // 16, fill_o, 0)

    pltpu.sync_copy(zbuf, deg_sh.at[pl.ds(s * DEG_SLICE, DEG_SLICE)])
    plsc.subcore_barrier()

    nj = jnp.where(w == NW - 1, TAIL, CHPW)

    def body(j, _):
        pltpu.async_copy(ones_v, deg_sh.at[eidx.at[j, 1]], sem, add=True)
        return 0

    lax.fori_loop(0, nj, body, 0)

    def drain(j, _):
        pltpu.make_async_copy(out_hbm.at[c, pl.ds(0, CHUNK)],
                              ones_v, sem).wait()
        return 0

    lax.fori_loop(0, nj, drain, 0)
    plsc.subcore_barrier()

    pltpu.sync_copy(deg_sh.at[pl.ds(s * DEG_SLICE, DEG_SLICE)],
                    out_hbm.at[c, pl.ds(s * DEG_SLICE, DEG_SLICE)])


@functools.partial(
    pl.kernel,
    out_type=jax.ShapeDtypeStruct((NC, N, D), jnp.float32),
    mesh=_MESH,
    scratch_types=[
        pltpu.VMEM_SHARED((ACC_ROWS, D), jnp.float32),
        pltpu.VMEM((HALF, 2, CHUNK), jnp.int32),
        pltpu.VMEM((CHUNK, D), jnp.float32),
        pltpu.VMEM((CHUNK, D), jnp.float32),
        pltpu.SemaphoreType.DMA,
        pltpu.SemaphoreType.DMA,
    ],
)
def _mp_sc(y_hbm, e3_hbm, out_hbm, acc_sh, eidx, rows0, rows1, sem0, sem1):
    c = lax.axis_index("c")
    s = lax.axis_index("s")
    w = c * NS + s

    def fill_z(r, _):
        for j in range(D // 16):
            rows0[r, pl.ds(j * 16, 16)] = jnp.zeros((16,), jnp.float32)
        return 0

    lax.fori_loop(0, CHUNK, fill_z, 0)

    for k in range(625 // CHUNK):
        pltpu.sync_copy(rows0, acc_sh.at[pl.ds(s * 625 + k * CHUNK, CHUNK)])
    pltpu.sync_copy(rows0.at[pl.ds(0, 625 % CHUNK)],
                    acc_sh.at[pl.ds(s * 625 + 625 - 625 % CHUNK, 625 % CHUNK)])

    plsc.subcore_barrier()

    def gather_start(j, buf, sem):
        for b in range(NB):
            pltpu.async_copy(y_hbm.at[eidx.at[j, 0, pl.ds(b * SUB, SUB)]],
                             buf.at[pl.ds(b * SUB, SUB)], sem)

    def gather_wait(buf, sem):
        pltpu.make_async_copy(y_hbm.at[pl.ds(0, CHUNK)], buf, sem).wait()

    def scat(j, buf):
        pltpu.sync_copy(buf, acc_sh.at[eidx.at[j, 1]], add=True)

    def make_body(half_len):
        def body(i, _):
            j0 = 2 * i
            gather_start(j0 + 1, rows1, sem1)
            gather_wait(rows0, sem0)
            scat(j0, rows0)

            @pl.when(j0 + 2 < half_len)
            def _():
                gather_start(j0 + 2, rows0, sem0)

            gather_wait(rows1, sem1)
            scat(j0 + 1, rows1)
            return 0
        return body

    def run(lo, half_len):
        pltpu.sync_copy(e3_hbm.at[pl.ds(lo, half_len)],
                        eidx.at[pl.ds(0, half_len)])
        gather_start(0, rows0, sem0)
        lax.fori_loop(0, half_len // 2, make_body(half_len), 0)

    @pl.when(w < NW - 1)
    def _():
        for h in range(CHPW // HALF):
            run(w * CHPW + h * HALF, HALF)

    @pl.when(w == NW - 1)
    def _():
        run((NW - 1) * CHPW, TAIL)

    plsc.subcore_barrier()

    @pl.when(s < NS - 1)
    def _():
        pltpu.sync_copy(acc_sh.at[pl.ds(s * WB_ROWS, WB_ROWS)],
                        out_hbm.at[c, pl.ds(s * WB_ROWS, WB_ROWS)])

    @pl.when(s == NS - 1)
    def _():
        pltpu.sync_copy(acc_sh.at[pl.ds((NS - 1) * WB_ROWS, WB_LAST)],
                        out_hbm.at[c, pl.ds((NS - 1) * WB_ROWS, WB_LAST)])


RB = 1000


def _tc_pre_body(deg_ref, x_ref, w_ref, y_ref):
    degb = deg_ref[...]
    dinv = lax.rsqrt(degb[0] + degb[1] + 1.0)
    y_ref[...] = lax.dot_general(
        x_ref[...], w_ref[...], (((1,), (1,)), ((), ())),
        preferred_element_type=jnp.float32) * dinv


def _tc_mid_body(acc_ref, y_ref, deg_ref, b_ref, w_ref, y2_ref):
    degb = deg_ref[...]
    dinv = lax.rsqrt(degb[0] + degb[1] + 1.0)
    accb = acc_ref[...]
    h = jnp.maximum((accb[0] + accb[1] + y_ref[...]) * dinv + b_ref[...], 0.0)
    y2_ref[...] = lax.dot_general(
        h, w_ref[...], (((1,), (1,)), ((), ())),
        preferred_element_type=jnp.float32) * dinv


def _tc_post_body(acc_ref, y_ref, deg_ref, b_ref, out_ref):
    degb = deg_ref[...]
    dinv = lax.rsqrt(degb[0] + degb[1] + 1.0)
    accb = acc_ref[...]
    out_ref[...] = (accb[0] + accb[1] + y_ref[...]) * dinv + b_ref[...]


def kernel(x, edge_index, W1, b1, W2, b2):
    e3 = edge_index.reshape(2, NCHUNKS, CHUNK).transpose(1, 0, 2)

    degp = _deg_sc(e3)
    deg3 = degp.reshape(NC, DEG_PAD, 1)

    y1 = pl.pallas_call(
        _tc_pre_body,
        grid=(N // RB,),
        in_specs=[
            pl.BlockSpec((NC, RB, 1), lambda i: (0, i, 0)),
            pl.BlockSpec((RB, D), lambda i: (i, 0)),
            pl.BlockSpec((D, D), lambda i: (0, 0)),
        ],
        out_specs=pl.BlockSpec((RB, D), lambda i: (i, 0)),
        out_shape=jax.ShapeDtypeStruct((N, D), jnp.float32),
    )(deg3, x, W1)

    acc1 = _mp_sc(y1, e3)

    y2 = pl.pallas_call(
        _tc_mid_body,
        grid=(N // RB,),
        in_specs=[
            pl.BlockSpec((NC, RB, D), lambda i: (0, i, 0)),
            pl.BlockSpec((RB, D), lambda i: (i, 0)),
            pl.BlockSpec((NC, RB, 1), lambda i: (0, i, 0)),
            pl.BlockSpec((1, D), lambda i: (0, 0)),
            pl.BlockSpec((D, D), lambda i: (0, 0)),
        ],
        out_specs=pl.BlockSpec((RB, D), lambda i: (i, 0)),
        out_shape=jax.ShapeDtypeStruct((N, D), jnp.float32),
    )(acc1, y1, deg3, b1.reshape(1, D), W2)

    acc2 = _mp_sc(y2, e3)

    out = pl.pallas_call(
        _tc_post_body,
        grid=(N // RB,),
        in_specs=[
            pl.BlockSpec((NC, RB, D), lambda i: (0, i, 0)),
            pl.BlockSpec((RB, D), lambda i: (i, 0)),
            pl.BlockSpec((NC, RB, 1), lambda i: (0, i, 0)),
            pl.BlockSpec((1, D), lambda i: (0, 0)),
        ],
        out_specs=pl.BlockSpec((RB, D), lambda i: (i, 0)),
        out_shape=jax.ShapeDtypeStruct((N, D), jnp.float32),
    )(acc2, y2, deg3, b2.reshape(1, D))

    return out


# speedup vs baseline: 1.0039x; 1.0039x over previous
import functools

import jax
import jax.numpy as jnp
from jax import lax
from jax.experimental import pallas as pl
from jax.experimental.pallas import tpu as pltpu
from jax.experimental.pallas import tpu_sc as plsc

N = 10000
E = 320000
D = 128

NC = 2
NS = 16
NW = NC * NS

CHUNK = 128
NB = 4
SUB = CHUNK // NB
NCHUNKS = E // CHUNK
CHPW = 80
HALF = CHPW // 2
TAIL = NCHUNKS - (NW - 1) * CHPW
ACC_ROWS = N
WB_ROWS = 632
WB_LAST = N - (NS - 1) * WB_ROWS

DEG_PAD = 10240
DEG_SLICE = DEG_PAD // NS

_MESH = plsc.VectorSubcoreMesh(core_axis_name="c", subcore_axis_name="s")


@functools.partial(
    pl.kernel,
    out_type=jax.ShapeDtypeStruct((NC, DEG_PAD), jnp.float32),
    mesh=_MESH,
    scratch_types=[
        pltpu.VMEM_SHARED((DEG_PAD,), jnp.float32),
        pltpu.VMEM((DEG_SLICE,), jnp.float32),
        pltpu.VMEM((CHUNK,), jnp.float32),
        pltpu.VMEM((CHPW, 2, CHUNK), jnp.int32),
        pltpu.SemaphoreType.DMA,
    ],
)
def _deg_sc(e3_hbm, out_hbm, deg_sh, zbuf, ones_v, eidx, sem):
    c = lax.axis_index("c")
    s = lax.axis_index("s")
    w = c * NS + s

    @pl.when(w < NW - 1)
    def _():
        pltpu.sync_copy(e3_hbm.at[pl.ds(w * CHPW, CHPW)], eidx)

    @pl.when(w == NW - 1)
    def _():
        pltpu.sync_copy(e3_hbm.at[pl.ds((NW - 1) * CHPW, TAIL)],
                        eidx.at[pl.ds(0, TAIL)])

    def fill_z(i, _):
        zbuf[pl.ds(i * 16, 16)] = jnp.zeros((16,), jnp.float32)
        return 0

    lax.fori_loop(0, DEG_SLICE // 16, fill_z, 0)

    def fill_o(i, _):
        ones_v[pl.ds(i * 16, 16)] = jnp.ones((16,), jnp.float32)
        return 0

    lax.fori_loop(0, CHUNK // 16, fill_o, 0)

    pltpu.sync_copy(zbuf, deg_sh.at[pl.ds(s * DEG_SLICE, DEG_SLICE)])
    plsc.subcore_barrier()

    nj = jnp.where(w == NW - 1, TAIL, CHPW)

    def body(j, _):
        pltpu.async_copy(ones_v, deg_sh.at[eidx.at[j, 1]], sem, add=True)
        return 0

    lax.fori_loop(0, nj, body, 0)

    def drain(j, _):
        pltpu.make_async_copy(out_hbm.at[c, pl.ds(0, CHUNK)],
                              ones_v, sem).wait()
        return 0

    lax.fori_loop(0, nj, drain, 0)
    plsc.subcore_barrier()

    pltpu.sync_copy(deg_sh.at[pl.ds(s * DEG_SLICE, DEG_SLICE)],
                    out_hbm.at[c, pl.ds(s * DEG_SLICE, DEG_SLICE)])


@functools.partial(
    pl.kernel,
    out_type=jax.ShapeDtypeStruct((NC, N, D), jnp.float32),
    mesh=_MESH,
    scratch_types=[
        pltpu.VMEM_SHARED((ACC_ROWS, D), jnp.float32),
        pltpu.VMEM((HALF, 2, CHUNK), jnp.int32),
        pltpu.VMEM((CHUNK, D), jnp.float32),
        pltpu.VMEM((CHUNK, D), jnp.float32),
        pltpu.SemaphoreType.DMA,
        pltpu.SemaphoreType.DMA,
    ],
)
def _mp_sc(y_hbm, e3_hbm, out_hbm, acc_sh, eidx, rows0, rows1, sem0, sem1):
    c = lax.axis_index("c")
    s = lax.axis_index("s")
    w = c * NS + s

    def fill_z(r, _):
        for j in range(D // 16):
            rows0[r, pl.ds(j * 16, 16)] = jnp.zeros((16,), jnp.float32)
        return 0

    lax.fori_loop(0, CHUNK, fill_z, 0)

    for k in range(625 // CHUNK):
        pltpu.sync_copy(rows0, acc_sh.at[pl.ds(s * 625 + k * CHUNK, CHUNK)])
    pltpu.sync_copy(rows0.at[pl.ds(0, 625 % CHUNK)],
                    acc_sh.at[pl.ds(s * 625 + 625 - 625 % CHUNK, 625 % CHUNK)])

    plsc.subcore_barrier()

    def gather_start(j, buf, sem):
        for b in range(NB):
            pltpu.async_copy(y_hbm.at[eidx.at[j, 0, pl.ds(b * SUB, SUB)]],
                             buf.at[pl.ds(b * SUB, SUB)], sem)

    def gather_wait(buf, sem):
        pltpu.make_async_copy(y_hbm.at[pl.ds(0, CHUNK)], buf, sem).wait()

    def scat(j, buf):
        pltpu.sync_copy(buf, acc_sh.at[eidx.at[j, 1]], add=True)

    def make_body(half_len):
        def body(i, _):
            j0 = 2 * i
            gather_start(j0 + 1, rows1, sem1)
            gather_wait(rows0, sem0)
            scat(j0, rows0)

            @pl.when(j0 + 2 < half_len)
            def _():
                gather_start(j0 + 2, rows0, sem0)

            gather_wait(rows1, sem1)
            scat(j0 + 1, rows1)
            return 0
        return body

    def run(lo, half_len):
        pltpu.sync_copy(e3_hbm.at[pl.ds(lo, half_len)],
                        eidx.at[pl.ds(0, half_len)])
        gather_start(0, rows0, sem0)
        lax.fori_loop(0, half_len // 2, make_body(half_len), 0)

    @pl.when(w < NW - 1)
    def _():
        for h in range(CHPW // HALF):
            run(w * CHPW + h * HALF, HALF)

    @pl.when(w == NW - 1)
    def _():
        run((NW - 1) * CHPW, TAIL)

    plsc.subcore_barrier()

    @pl.when(s < NS - 1)
    def _():
        pltpu.sync_copy(acc_sh.at[pl.ds(s * WB_ROWS, WB_ROWS)],
                        out_hbm.at[c, pl.ds(s * WB_ROWS, WB_ROWS)])

    @pl.when(s == NS - 1)
    def _():
        pltpu.sync_copy(acc_sh.at[pl.ds((NS - 1) * WB_ROWS, WB_LAST)],
                        out_hbm.at[c, pl.ds((NS - 1) * WB_ROWS, WB_LAST)])


RB = 1000


def _tc_mm_body(x_ref, w_ref, y_ref):
    y_ref[...] = lax.dot_general(
        x_ref[...], w_ref[...], (((1,), (1,)), ((), ())),
        preferred_element_type=jnp.float32)


def _tc_scale_body(deg_ref, xw_ref, y_ref):
    degb = deg_ref[...]
    dinv = lax.rsqrt(degb[0] + degb[1] + 1.0)
    y_ref[...] = xw_ref[...] * dinv


def _tc_mid_body(acc_ref, y_ref, deg_ref, b_ref, w_ref, y2_ref):
    degb = deg_ref[...]
    dinv = lax.rsqrt(degb[0] + degb[1] + 1.0)
    accb = acc_ref[...]
    h = jnp.maximum((accb[0] + accb[1] + y_ref[...]) * dinv + b_ref[...], 0.0)
    y2_ref[...] = lax.dot_general(
        h, w_ref[...], (((1,), (1,)), ((), ())),
        preferred_element_type=jnp.float32) * dinv


def _tc_post_body(acc_ref, y_ref, deg_ref, b_ref, out_ref):
    degb = deg_ref[...]
    dinv = lax.rsqrt(degb[0] + degb[1] + 1.0)
    accb = acc_ref[...]
    out_ref[...] = (accb[0] + accb[1] + y_ref[...]) * dinv + b_ref[...]


def kernel(x, edge_index, W1, b1, W2, b2):
    e3 = edge_index.reshape(2, NCHUNKS, CHUNK).transpose(1, 0, 2)

    degp = _deg_sc(e3)
    deg3 = degp.reshape(NC, DEG_PAD, 1)

    xw1 = pl.pallas_call(
        _tc_mm_body,
        grid=(N // RB,),
        in_specs=[
            pl.BlockSpec((RB, D), lambda i: (i, 0)),
            pl.BlockSpec((D, D), lambda i: (0, 0)),
        ],
        out_specs=pl.BlockSpec((RB, D), lambda i: (i, 0)),
        out_shape=jax.ShapeDtypeStruct((N, D), jnp.float32),
    )(x, W1)

    y1 = pl.pallas_call(
        _tc_scale_body,
        grid=(N // RB,),
        in_specs=[
            pl.BlockSpec((NC, RB, 1), lambda i: (0, i, 0)),
            pl.BlockSpec((RB, D), lambda i: (i, 0)),
        ],
        out_specs=pl.BlockSpec((RB, D), lambda i: (i, 0)),
        out_shape=jax.ShapeDtypeStruct((N, D), jnp.float32),
    )(deg3, xw1)

    acc1 = _mp_sc(y1, e3)

    y2 = pl.pallas_call(
        _tc_mid_body,
        grid=(N // RB,),
        in_specs=[
            pl.BlockSpec((NC, RB, D), lambda i: (0, i, 0)),
            pl.BlockSpec((RB, D), lambda i: (i, 0)),
            pl.BlockSpec((NC, RB, 1), lambda i: (0, i, 0)),
            pl.BlockSpec((1, D), lambda i: (0, 0)),
            pl.BlockSpec((D, D), lambda i: (0, 0)),
        ],
        out_specs=pl.BlockSpec((RB, D), lambda i: (i, 0)),
        out_shape=jax.ShapeDtypeStruct((N, D), jnp.float32),
    )(acc1, y1, deg3, b1.reshape(1, D), W2)

    acc2 = _mp_sc(y2, e3)

    out = pl.pallas_call(
        _tc_post_body,
        grid=(N // RB,),
        in_specs=[
            pl.BlockSpec((NC, RB, D), lambda i: (0, i, 0)),
            pl.BlockSpec((RB, D), lambda i: (i, 0)),
            pl.BlockSpec((NC, RB, 1), lambda i: (0, i, 0)),
            pl.BlockSpec((1, D), lambda i: (0, 0)),
        ],
        out_specs=pl.BlockSpec((RB, D), lambda i: (i, 0)),
        out_shape=jax.ShapeDtypeStruct((N, D), jnp.float32),
    )(acc2, y2, deg3, b2.reshape(1, D))

    return out
